# Initial kernel scaffold; baseline (speedup 1.0000x reference)
import jax
import jax.numpy as jnp
from jax.experimental import pallas as pl


def kernel(x, edge_index, att):
    raise NotImplementedError("write your pallas kernel here")


# speedup vs baseline: 29.8513x; 29.8513x over previous
import functools

import jax
import jax.numpy as jnp
from jax import lax
from jax.experimental import pallas as pl
from jax.experimental.pallas import tpu as pltpu
from jax.experimental.pallas import tpu_sc as plsc

_NEG_SLOPE = 0.2
_NC, _NS, _L = 2, 16, 16
_NW = _NC * _NS
_NEG_BIG = -1.0e30


def _mesh():
    return plsc.VectorSubcoreMesh(
        core_axis_name="c", subcore_axis_name="s",
        num_cores=_NC, num_subcores=_NS)


_SC_PARAMS = pltpu.CompilerParams(needs_layout_passes=False, use_tc_tiling_on_sc=False)


def _wid():
    return lax.axis_index("s") * _NC + lax.axis_index("c")


def _scores_tc(x, att2):
    n, _ = x.shape

    def body(x_ref, a_ref, sr_ref, sc_ref):
        xv = x_ref[...]
        sr_ref[...] = jnp.sum(xv * a_ref[0:1, :], axis=1, keepdims=True)
        sc_ref[...] = jnp.sum(xv * a_ref[1:2, :], axis=1, keepdims=True)

    sr, sc = pl.pallas_call(
        body,
        out_shape=[jax.ShapeDtypeStruct((n, 1), jnp.float32),
                   jax.ShapeDtypeStruct((n, 1), jnp.float32)],
    )(x, att2)
    return sr.reshape(n), sc.reshape(n)


def _edge_w(srv, scv, rowv, colv, off):
    r = rowv[pl.ds(off, _L)]
    c = colv[pl.ds(off, _L)]
    w = plsc.load_gather(srv, [r]) + plsc.load_gather(scv, [c])
    return r, jnp.where(w >= 0.0, w, _NEG_SLOPE * w)


def _seg_max_partials(row, col, sr, sc, n, n_pad, ew_per, blk):
    @functools.partial(
        pl.kernel,
        out_type=jax.ShapeDtypeStruct((_NW, n_pad), jnp.float32),
        mesh=_mesh(),
        compiler_params=_SC_PARAMS,
        scratch_types=[
            pltpu.VMEM((blk,), jnp.int32),
            pltpu.VMEM((blk,), jnp.int32),
            pltpu.VMEM((n,), jnp.float32),
            pltpu.VMEM((n,), jnp.float32),
            pltpu.VMEM((n_pad,), jnp.float32),
        ],
    )
    def k(row_h, col_h, sr_h, sc_h, pm_h, rowv, colv, srv, scv, mloc):
        wid = _wid()
        base = wid * ew_per
        pltpu.sync_copy(sr_h, srv)
        pltpu.sync_copy(sc_h, scv)

        def init(i, carry):
            mloc[pl.ds(pl.multiple_of(i * _L, _L), _L)] = jnp.full(
                (_L,), _NEG_BIG, jnp.float32)
            return carry
        lax.fori_loop(0, n_pad // _L, init, 0)

        for b in range(ew_per // blk):
            pltpu.sync_copy(row_h.at[pl.ds(base + b * blk, blk)], rowv)
            pltpu.sync_copy(col_h.at[pl.ds(base + b * blk, blk)], colv)

            def chunk(i, carry):
                off = pl.multiple_of(i * _L, _L)
                r, w = _edge_w(srv, scv, rowv, colv, off)
                cur = plsc.load_gather(mloc, [r])

                def cond(a):
                    return jnp.any(a)

                def body(a):
                    plsc.store_scatter(mloc, [r], w, mask=a)
                    cur2 = plsc.load_gather(mloc, [r])
                    return a & (cur2 < w)

                lax.while_loop(cond, body, cur < w)
                return carry
            lax.fori_loop(0, blk // _L, chunk, 0)

        pltpu.sync_copy(mloc, pm_h.at[wid])

    return k(row, col, sr, sc)


def _reduce_partials(pm, n_pad, is_max):
    cb = n_pad // _NW

    @functools.partial(
        pl.kernel,
        out_type=jax.ShapeDtypeStruct((n_pad,), jnp.float32),
        mesh=_mesh(),
        compiler_params=_SC_PARAMS,
        scratch_types=[
            pltpu.VMEM((_NW, cb), jnp.float32),
            pltpu.VMEM((cb,), jnp.float32),
            pltpu.SemaphoreType.DMA,
        ],
    )
    def k(pm_h, out_h, pv, ov, sem):
        wid = _wid()
        c0 = wid * cb
        copies = [pltpu.async_copy(pm_h.at[t, pl.ds(c0, cb)], pv.at[t], sem)
                  for t in range(_NW)]
        for cp in copies:
            cp.wait()

        def body(j, carry):
            off = pl.multiple_of(j * _L, _L)
            acc = pv[0, pl.ds(off, _L)]
            for t in range(1, _NW):
                nxt = pv[t, pl.ds(off, _L)]
                acc = jnp.maximum(acc, nxt) if is_max else acc + nxt
            ov[pl.ds(off, _L)] = acc
            return carry
        lax.fori_loop(0, cb // _L, body, 0)
        pltpu.sync_copy(ov, out_h.at[pl.ds(c0, cb)])

    return k(pm)


def _seg_sum_partials(row, col, sr, sc, m, n, n_pad, ew_per, blk):
    @functools.partial(
        pl.kernel,
        out_type=jax.ShapeDtypeStruct((_NW, n_pad), jnp.float32),
        mesh=_mesh(),
        compiler_params=_SC_PARAMS,
        scratch_types=[
            pltpu.VMEM((blk,), jnp.int32),
            pltpu.VMEM((blk,), jnp.int32),
            pltpu.VMEM((n,), jnp.float32),
            pltpu.VMEM((n,), jnp.float32),
            pltpu.VMEM((n_pad,), jnp.float32),
            pltpu.VMEM((n_pad,), jnp.float32),
        ],
    )
    def k(row_h, col_h, sr_h, sc_h, m_h, pd_h, rowv, colv, srv, scv, mv, dloc):
        wid = _wid()
        base = wid * ew_per
        pltpu.sync_copy(sr_h, srv)
        pltpu.sync_copy(sc_h, scv)
        pltpu.sync_copy(m_h, mv)

        def init(i, carry):
            dloc[pl.ds(pl.multiple_of(i * _L, _L), _L)] = jnp.zeros(
                (_L,), jnp.float32)
            return carry
        lax.fori_loop(0, n_pad // _L, init, 0)

        for b in range(ew_per // blk):
            pltpu.sync_copy(row_h.at[pl.ds(base + b * blk, blk)], rowv)
            pltpu.sync_copy(col_h.at[pl.ds(base + b * blk, blk)], colv)

            def chunk(i, carry):
                off = pl.multiple_of(i * _L, _L)
                r, w = _edge_w(srv, scv, rowv, colv, off)
                ew = jnp.exp(w - plsc.load_gather(mv, [r]))
                plsc.addupdate_scatter(dloc, [r], ew)
                return carry
            lax.fori_loop(0, blk // _L, chunk, 0)

        pltpu.sync_copy(dloc, pd_h.at[wid])

    return k(row, col, sr, sc, m)


def _normalize(row, col, sr, sc, m, d, n, n_pad, e, ew_per, blk):
    @functools.partial(
        pl.kernel,
        out_type=jax.ShapeDtypeStruct((e,), jnp.float32),
        mesh=_mesh(),
        compiler_params=_SC_PARAMS,
        scratch_types=[
            pltpu.VMEM((blk,), jnp.int32),
            pltpu.VMEM((blk,), jnp.int32),
            pltpu.VMEM((n,), jnp.float32),
            pltpu.VMEM((n,), jnp.float32),
            pltpu.VMEM((n_pad,), jnp.float32),
            pltpu.VMEM((n_pad,), jnp.float32),
            pltpu.VMEM((blk,), jnp.float32),
        ],
    )
    def k(row_h, col_h, sr_h, sc_h, m_h, d_h, attr_h,
          rowv, colv, srv, scv, mv, dv, av):
        wid = _wid()
        base = wid * ew_per
        pltpu.sync_copy(sr_h, srv)
        pltpu.sync_copy(sc_h, scv)
        pltpu.sync_copy(m_h, mv)
        pltpu.sync_copy(d_h, dv)

        for b in range(ew_per // blk):
            pltpu.sync_copy(row_h.at[pl.ds(base + b * blk, blk)], rowv)
            pltpu.sync_copy(col_h.at[pl.ds(base + b * blk, blk)], colv)

            def chunk(i, carry):
                off = pl.multiple_of(i * _L, _L)
                r, w = _edge_w(srv, scv, rowv, colv, off)
                ew = jnp.exp(w - plsc.load_gather(mv, [r]))
                den = plsc.load_gather(dv, [r])
                av[pl.ds(off, _L)] = ew / jnp.maximum(den, 1e-16)
                return carry
            lax.fori_loop(0, blk // _L, chunk, 0)
            pltpu.sync_copy(av, attr_h.at[pl.ds(base + b * blk, blk)])

    return k(row, col, sr, sc, m, d)


def kernel(x, edge_index, att):
    n, dmodel = x.shape
    e = edge_index.shape[1]
    assert e % (_NW * _L) == 0
    ew_per = e // _NW
    blk = 2000
    assert ew_per % blk == 0 and blk % _L == 0
    n_pad = -(-n // (_NW * _L)) * (_NW * _L)

    att2 = att.reshape(2, dmodel)
    sr, sc = _scores_tc(x, att2)
    row = edge_index[0]
    col = edge_index[1]

    pm = _seg_max_partials(row, col, sr, sc, n, n_pad, ew_per, blk)
    m = _reduce_partials(pm, n_pad, is_max=True)
    pd = _seg_sum_partials(row, col, sr, sc, m, n, n_pad, ew_per, blk)
    d = _reduce_partials(pd, n_pad, is_max=False)
    attr = _normalize(row, col, sr, sc, m, d, n, n_pad, e, ew_per, blk)
    return (x, edge_index, attr)


# speedup vs baseline: 36.1168x; 1.2099x over previous
import functools

import jax
import jax.numpy as jnp
from jax import lax
from jax.experimental import pallas as pl
from jax.experimental.pallas import tpu as pltpu
from jax.experimental.pallas import tpu_sc as plsc

_NEG_SLOPE = 0.2
_NC, _NS, _L = 2, 16, 16
_NW = _NC * _NS
_NEG_BIG = -1.0e30


def _mesh():
    return plsc.VectorSubcoreMesh(
        core_axis_name="c", subcore_axis_name="s",
        num_cores=_NC, num_subcores=_NS)


_SC_PARAMS = pltpu.CompilerParams(
    needs_layout_passes=False, use_tc_tiling_on_sc=False)


def _scores_tc(x, att2):
    n, _ = x.shape

    def body(x_ref, a_ref, sr_ref, sc_ref):
        xv = x_ref[...]
        sr_ref[...] = jnp.sum(xv * a_ref[0:1, :], axis=1, keepdims=True)
        sc_ref[...] = jnp.sum(xv * a_ref[1:2, :], axis=1, keepdims=True)

    sr, sc = pl.pallas_call(
        body,
        out_shape=[jax.ShapeDtypeStruct((n, 1), jnp.float32),
                   jax.ShapeDtypeStruct((n, 1), jnp.float32)],
    )(x, att2)
    return sr.reshape(n), sc.reshape(n)


def _edge_w(srv, scv, rowv, colv, off):
    r = rowv[pl.ds(off, _L)]
    c = colv[pl.ds(off, _L)]
    w = plsc.load_gather(srv, [r]) + plsc.load_gather(scv, [c])
    return r, jnp.where(w >= 0.0, w, _NEG_SLOPE * w)


def _fill(ref, n, value):
    v = jnp.full((_L,), value, jnp.float32)

    def body(i, carry):
        base = pl.multiple_of(i * (4 * _L), _L)
        for u in range(4):
            ref[pl.ds(base + u * _L, _L)] = v
        return carry
    lax.fori_loop(0, n // (4 * _L), body, 0)


def _stage_edges(ei_h, rowv, colv, base, blk, sem):
    cps = [pltpu.async_copy(ei_h.at[0, pl.ds(base, blk)], rowv, sem),
           pltpu.async_copy(ei_h.at[1, pl.ds(base, blk)], colv, sem)]
    for cp in cps:
        cp.wait()


def _sc_reduce_tail(loc, sh, red, outv, out_h, cid, sid, n_pad, sem, is_max):
    cbs = n_pad // _NS
    pltpu.sync_copy(loc, sh.at[sid])
    plsc.subcore_barrier()
    c0 = sid * cbs
    cps = [pltpu.async_copy(sh.at[t, pl.ds(c0, cbs)], red.at[t], sem)
           for t in range(_NS)]
    for cp in cps:
        cp.wait()

    def body(j, carry):
        off = pl.multiple_of(j * _L, _L)
        acc = red[0, pl.ds(off, _L)]
        for t in range(1, _NS):
            nxt = red[t, pl.ds(off, _L)]
            acc = jnp.maximum(acc, nxt) if is_max else acc + nxt
        outv[pl.ds(off, _L)] = acc
        return carry
    lax.fori_loop(0, cbs // _L, body, 0)
    pltpu.sync_copy(outv, out_h.at[cid, pl.ds(c0, cbs)])


def _merge2(tmp, dst, n_pad, is_max):
    def body(j, carry):
        base = pl.multiple_of(j * (4 * _L), _L)
        for u in range(4):
            off = base + u * _L
            a = tmp[0, pl.ds(off, _L)]
            b = tmp[1, pl.ds(off, _L)]
            dst[pl.ds(off, _L)] = jnp.maximum(a, b) if is_max else a + b
        return carry
    lax.fori_loop(0, n_pad // (4 * _L), body, 0)


def _seg_max_partials(ei, sr, sc, n, n_pad, ew_per, blk):
    @functools.partial(
        pl.kernel,
        out_type=jax.ShapeDtypeStruct((_NC, n_pad), jnp.float32),
        mesh=_mesh(),
        compiler_params=_SC_PARAMS,
        scratch_types=[
            pltpu.VMEM((blk,), jnp.int32),
            pltpu.VMEM((blk,), jnp.int32),
            pltpu.VMEM((n,), jnp.float32),
            pltpu.VMEM((n,), jnp.float32),
            pltpu.VMEM((n_pad,), jnp.float32),
            pltpu.VMEM_SHARED((_NS, n_pad), jnp.float32),
            pltpu.VMEM((_NS, n_pad // _NS), jnp.float32),
            pltpu.VMEM((n_pad // _NS,), jnp.float32),
            pltpu.SemaphoreType.DMA,
        ],
    )
    def k(ei_h, sr_h, sc_h, pm_h, rowv, colv, srv, scv, mloc, msh, red, outv,
          sem):
        cid = lax.axis_index("c")
        sid = lax.axis_index("s")
        wid = sid * _NC + cid
        pltpu.sync_copy(sr_h, srv)
        pltpu.sync_copy(sc_h, scv)
        _fill(mloc, n_pad, _NEG_BIG)

        for b in range(ew_per // blk):
            _stage_edges(ei_h, rowv, colv, wid * ew_per + b * blk, blk, sem)

            def chunk(i, carry):
                off = pl.multiple_of(i * _L, _L)
                r, w = _edge_w(srv, scv, rowv, colv, off)
                cur = plsc.load_gather(mloc, [r])

                def cond(a):
                    return jnp.any(a)

                def body(a):
                    plsc.store_scatter(mloc, [r], w, mask=a)
                    cur2 = plsc.load_gather(mloc, [r])
                    return a & (cur2 < w)

                lax.while_loop(cond, body, cur < w)
                return carry
            lax.fori_loop(0, blk // _L, chunk, 0)

        _sc_reduce_tail(mloc, msh, red, outv, pm_h, cid, sid, n_pad, sem,
                        is_max=True)

    return k(ei, sr, sc)


def _seg_sum_partials(ei, sr, sc, pm, n, n_pad, ew_per, blk):
    @functools.partial(
        pl.kernel,
        out_type=[jax.ShapeDtypeStruct((_NC, n_pad), jnp.float32),
                  jax.ShapeDtypeStruct((n_pad,), jnp.float32)],
        mesh=_mesh(),
        compiler_params=_SC_PARAMS,
        scratch_types=[
            pltpu.VMEM((blk,), jnp.int32),
            pltpu.VMEM((blk,), jnp.int32),
            pltpu.VMEM((n,), jnp.float32),
            pltpu.VMEM((n,), jnp.float32),
            pltpu.VMEM((_NC, n_pad), jnp.float32),
            pltpu.VMEM((n_pad,), jnp.float32),
            pltpu.VMEM((n_pad,), jnp.float32),
            pltpu.VMEM_SHARED((_NS, n_pad), jnp.float32),
            pltpu.VMEM((_NS, n_pad // _NS), jnp.float32),
            pltpu.VMEM((n_pad // _NS,), jnp.float32),
            pltpu.SemaphoreType.DMA,
        ],
    )
    def k(ei_h, sr_h, sc_h, pm_h, pd_h, m_h, rowv, colv, srv, scv, tmp, mv,
          dloc, msh, red, outv, sem):
        cid = lax.axis_index("c")
        sid = lax.axis_index("s")
        wid = sid * _NC + cid
        cps = [pltpu.async_copy(pm_h.at[t], tmp.at[t], sem)
               for t in range(_NC)]
        pltpu.sync_copy(sr_h, srv)
        pltpu.sync_copy(sc_h, scv)
        for cp in cps:
            cp.wait()
        _merge2(tmp, mv, n_pad, is_max=True)
        _fill(dloc, n_pad, 0.0)
        cb = n_pad // _NW
        pltpu.sync_copy(mv.at[pl.ds(wid * cb, cb)],
                        m_h.at[pl.ds(wid * cb, cb)])

        for b in range(ew_per // blk):
            _stage_edges(ei_h, rowv, colv, wid * ew_per + b * blk, blk, sem)

            def chunk(i, carry):
                off = pl.multiple_of(i * _L, _L)
                r, w = _edge_w(srv, scv, rowv, colv, off)
                ew = jnp.exp(w - plsc.load_gather(mv, [r]))
                plsc.addupdate_scatter(dloc, [r], ew)
                return carry
            lax.fori_loop(0, blk // _L, chunk, 0)

        _sc_reduce_tail(dloc, msh, red, outv, pd_h, cid, sid, n_pad, sem,
                        is_max=False)

    return k(ei, sr, sc, pm)


def _normalize(ei, sr, sc, m, pd, n, n_pad, e, ew_per, blk):
    @functools.partial(
        pl.kernel,
        out_type=jax.ShapeDtypeStruct((e,), jnp.float32),
        mesh=_mesh(),
        compiler_params=_SC_PARAMS,
        scratch_types=[
            pltpu.VMEM((blk,), jnp.int32),
            pltpu.VMEM((blk,), jnp.int32),
            pltpu.VMEM((n,), jnp.float32),
            pltpu.VMEM((n,), jnp.float32),
            pltpu.VMEM((n_pad,), jnp.float32),
            pltpu.VMEM((_NC, n_pad), jnp.float32),
            pltpu.VMEM((n_pad,), jnp.float32),
            pltpu.VMEM((blk,), jnp.float32),
            pltpu.SemaphoreType.DMA,
        ],
    )
    def k(ei_h, sr_h, sc_h, m_h, pd_h, attr_h,
          rowv, colv, srv, scv, mv, tmp, dv, av, sem):
        cid = lax.axis_index("c")
        sid = lax.axis_index("s")
        wid = sid * _NC + cid
        cps = [pltpu.async_copy(pd_h.at[t], tmp.at[t], sem)
               for t in range(_NC)]
        pltpu.sync_copy(sr_h, srv)
        pltpu.sync_copy(sc_h, scv)
        pltpu.sync_copy(m_h, mv)
        for cp in cps:
            cp.wait()
        _merge2(tmp, dv, n_pad, is_max=False)

        for b in range(ew_per // blk):
            base = wid * ew_per + b * blk
            _stage_edges(ei_h, rowv, colv, base, blk, sem)

            def chunk(i, carry):
                off = pl.multiple_of(i * _L, _L)
                r, w = _edge_w(srv, scv, rowv, colv, off)
                ew = jnp.exp(w - plsc.load_gather(mv, [r]))
                den = plsc.load_gather(dv, [r])
                av[pl.ds(off, _L)] = ew / jnp.maximum(den, 1e-16)
                return carry
            lax.fori_loop(0, blk // _L, chunk, 0)
            pltpu.sync_copy(av, attr_h.at[pl.ds(base, blk)])

    return k(ei, sr, sc, m, pd)


def kernel(x, edge_index, att):
    n, dmodel = x.shape
    e = edge_index.shape[1]
    assert e % (_NW * _L) == 0
    ew_per = e // _NW
    blk = 2000
    assert ew_per % blk == 0 and blk % _L == 0
    n_pad = -(-n // (_NW * _L)) * (_NW * _L)

    att2 = att.reshape(2, dmodel)
    sr, sc = _scores_tc(x, att2)

    pm = _seg_max_partials(edge_index, sr, sc, n, n_pad, ew_per, blk)
    pd, m = _seg_sum_partials(edge_index, sr, sc, pm, n, n_pad, ew_per, blk)
    attr = _normalize(edge_index, sr, sc, m, pd, n, n_pad, e, ew_per, blk)
    return (x, edge_index, attr)
